# baseline (speedup 1.0000x reference)
import functools

import jax
import jax.numpy as jnp
from jax import lax
from jax.experimental import pallas as pl
from jax.experimental.pallas import tpu as pltpu
from jax.experimental.pallas import tpu_sc as plsc

N = 10000
E = 320000
INC = 128
OUTC = 128
HID = 64
EPS = 1e-5

NC = 2
NS = 16
NW = NC * NS
LANES = 16

EX = 128000
EY = 128000
EZ = E - EX - EY
GCH = 200
SCH = 640
RPT = OUTC // NS
NP = 10240

PR = 1000
BE = 1280

_SC_MESH = plsc.VectorSubcoreMesh(
    core_axis_name="c", subcore_axis_name="s", num_cores=NC, num_subcores=NS
)


def _prep_body(x_ref, gamma_ref, beta_ref, wd_ref, wb_ref, b1_ref,
               t_ref, scale_ref, shift_ref):
    i = pl.program_id(0)

    @pl.when(i == 0)
    def _():
        xv = x_ref[...]
        mean = jnp.mean(xv, axis=0, keepdims=True)
        var = jnp.mean((xv - mean) ** 2, axis=0, keepdims=True)
        scale = gamma_ref[...] * lax.rsqrt(var + EPS)
        scale_ref[...] = scale
        shift_ref[...] = beta_ref[...] - mean * scale

    rows = x_ref[pl.ds(i * PR, PR), :]
    xb = rows * scale_ref[...] + shift_ref[...]
    a = (jnp.dot(xb, wd_ref[...], preferred_element_type=jnp.float32)
         + b1_ref[...])
    b = jnp.dot(xb, wb_ref[...], preferred_element_type=jnp.float32)
    t_ref[...] = jnp.concatenate([a, b], axis=-1)


def _prep(x, gamma, beta, wdT, wbT, b1):
    return pl.pallas_call(
        _prep_body,
        grid=(N // PR,),
        in_specs=[
            pl.BlockSpec((N, INC), lambda i: (0, 0)),
            pl.BlockSpec((1, INC), lambda i: (0, 0)),
            pl.BlockSpec((1, INC), lambda i: (0, 0)),
            pl.BlockSpec((INC, HID), lambda i: (0, 0)),
            pl.BlockSpec((INC, HID), lambda i: (0, 0)),
            pl.BlockSpec((1, HID), lambda i: (0, 0)),
        ],
        out_specs=pl.BlockSpec((PR, 2 * HID), lambda i: (i, 0)),
        out_shape=jax.ShapeDtypeStruct((N, 2 * HID), jnp.float32),
        scratch_shapes=[
            pltpu.VMEM((1, INC), jnp.float32),
            pltpu.VMEM((1, INC), jnp.float32),
        ],
    )(x, gamma, beta, wdT, wbT, b1)


def _make_gather(ne):
    npairs = ne // 2
    epw = npairs // NW
    nch = epw // GCH
    assert nch * GCH * NW * 2 == ne and (epw % 8) == 0

    @functools.partial(
        pl.kernel,
        out_type=jax.ShapeDtypeStruct((npairs, 2 * HID), jnp.float32),
        mesh=_SC_MESH,
        scratch_types=[
            pltpu.VMEM((GCH,), jnp.int32),
            pltpu.VMEM((GCH,), jnp.int32),
            pltpu.VMEM((GCH,), jnp.int32),
            pltpu.VMEM((GCH,), jnp.int32),
            pltpu.VMEM((GCH, 2 * HID), jnp.float32),
            pltpu.VMEM((GCH, 2 * HID), jnp.float32),
            pltpu.VMEM((GCH, 2 * HID), jnp.float32),
            pltpu.VMEM((GCH, 2 * HID), jnp.float32),
            pltpu.SemaphoreType.DMA,
            pltpu.SemaphoreType.DMA,
            pltpu.SemaphoreType.DMA,
        ],
    )
    def gather(t_hbm, dst_hbm, src_hbm, h_hbm,
               d0, s0, d1, s1, td0, ts0, td1, ts1, semi, semg0, semg1):
        wid = lax.axis_index("s") * NC + lax.axis_index("c")
        base = wid * epw

        def chunk(c, _):
            off = base + c * GCH
            ci0 = pltpu.async_copy(dst_hbm.at[pl.ds(off, GCH)], d0, semi)
            ci1 = pltpu.async_copy(src_hbm.at[pl.ds(off, GCH)], s0, semi)
            ci2 = pltpu.async_copy(dst_hbm.at[pl.ds(off + npairs, GCH)], d1,
                                   semi)
            ci3 = pltpu.async_copy(src_hbm.at[pl.ds(off + npairs, GCH)], s1,
                                   semi)
            ci0.wait()
            ci1.wait()
            ci2.wait()
            ci3.wait()
            g0d = pltpu.async_copy(t_hbm.at[d0], td0, semg0)
            g0s = pltpu.async_copy(t_hbm.at[s0], ts0, semg0)
            g1d = pltpu.async_copy(t_hbm.at[d1], td1, semg1)
            g1s = pltpu.async_copy(t_hbm.at[s1], ts1, semg1)
            g0d.wait()
            g0s.wait()

            def row0(e, _):
                for j in range(HID // LANES):
                    va = td0[e, pl.ds(j * LANES, LANES)]
                    vb = ts0[e, pl.ds(HID + j * LANES, LANES)]
                    td0[e, pl.ds(j * LANES, LANES)] = jnp.maximum(va + vb, 0.0)
                return 0

            lax.fori_loop(0, GCH, row0, 0)
            g1d.wait()
            g1s.wait()

            def row1(e, _):
                for j in range(HID // LANES):
                    va = td1[e, pl.ds(j * LANES, LANES)]
                    vb = ts1[e, pl.ds(HID + j * LANES, LANES)]
                    td0[e, pl.ds(HID + j * LANES, LANES)] = (
                        jnp.maximum(va + vb, 0.0))
                return 0

            lax.fori_loop(0, GCH, row1, 0)
            pltpu.sync_copy(td0, h_hbm.at[pl.ds(off, GCH)])
            return 0

        lax.fori_loop(0, nch, chunk, 0)

    return gather


def _msg_body(h_ref, wc_ref, m_ref):
    m = jnp.dot(h_ref[...], wc_ref[...], preferred_element_type=jnp.float32)
    m_ref[0] = m[:, :OUTC].T
    m_ref[1] = m[:, OUTC:].T


def _msg(h2, wcomb):
    npairs = h2.shape[0]
    return pl.pallas_call(
        _msg_body,
        grid=(npairs // BE,),
        in_specs=[
            pl.BlockSpec((BE, 2 * HID), lambda i: (i, 0)),
            pl.BlockSpec((2 * HID, 2 * OUTC), lambda i: (0, 0)),
        ],
        out_specs=pl.BlockSpec((2, OUTC, BE), lambda i: (0, 0, i)),
        out_shape=jax.ShapeDtypeStruct((2, OUTC, npairs), jnp.float32),
    )(h2, wcomb)


def _make_scatter(ne):
    nsc = ne // 2
    npair2 = nsc // (2 * SCH)
    assert npair2 * 2 * SCH == nsc

    @functools.partial(
        pl.kernel,
        out_type=jax.ShapeDtypeStruct((2, OUTC, NP), jnp.float32),
        mesh=_SC_MESH,
        scratch_types=[
            pltpu.VMEM((RPT, NP), jnp.float32),
            pltpu.VMEM((SCH,), jnp.int32),
            pltpu.VMEM((SCH,), jnp.int32),
            pltpu.VMEM((RPT, SCH), jnp.float32),
            pltpu.VMEM((RPT, SCH), jnp.float32),
            pltpu.SemaphoreType.DMA,
            pltpu.SemaphoreType.DMA,
        ],
        compiler_params=pltpu.CompilerParams(needs_layout_passes=False),
    )
    def scatter(mt_hbm, dst_hbm, out_hbm,
                tab, ibuf0, ibuf1, mbuf0, mbuf1, semA, semB):
        cid = lax.axis_index("c")
        sid = lax.axis_index("s")
        r0 = sid * RPT

        neg_inf = jnp.full((LANES,), -jnp.inf, jnp.float32)

        def initk(k, _):
            for r in range(RPT):
                tab[r, pl.ds(k * LANES, LANES)] = neg_inf
            return 0

        lax.fori_loop(0, NP // LANES, initk, 0)

        ibufs = (ibuf0, ibuf1)
        mbufs = (mbuf0, mbuf1)
        sems = (semA, semB)

        def consume(ibuf, mbuf):
            def group(g, _):
                idx = ibuf[pl.ds(g * LANES, LANES)]
                _, last = plsc.scan_count(idx)
                all_distinct = jnp.all(last)

                def fast():
                    for r in range(RPT):
                        rvec = jnp.full((LANES,), r, jnp.int32)
                        val = mbuf[r, pl.ds(g * LANES, LANES)]
                        cur = plsc.load_gather(tab, [rvec, idx])
                        plsc.store_scatter(tab, [rvec, idx],
                                           jnp.maximum(cur, val))

                def slow():
                    for r in range(RPT):
                        rvec = jnp.full((LANES,), r, jnp.int32)
                        val = mbuf[r, pl.ds(g * LANES, LANES)]

                        def attempt(_):
                            cur = plsc.load_gather(tab, [rvec, idx])
                            need = val > cur
                            plsc.store_scatter(tab, [rvec, idx], val,
                                               mask=need)
                            cur2 = plsc.load_gather(tab, [rvec, idx])
                            return jnp.any(val > cur2)

                        lax.while_loop(lambda p: p, attempt, attempt(True))

                lax.cond(all_distinct, fast, slow)
                return 0

            lax.fori_loop(0, SCH // LANES, group, 0)

        def chunk2(cc, _):
            handles = []
            for sl in range(2):
                off = (2 * cc + sl) * SCH
                handles.append((
                    pltpu.async_copy(
                        dst_hbm.at[pl.ds(cid * nsc + off, SCH)], ibufs[sl],
                        sems[sl]),
                    pltpu.async_copy(
                        mt_hbm.at[cid, pl.ds(r0, RPT), pl.ds(off, SCH)],
                        mbufs[sl], sems[sl]),
                ))
            for sl in range(2):
                handles[sl][0].wait()
                handles[sl][1].wait()
                consume(ibufs[sl], mbufs[sl])
            return 0

        lax.fori_loop(0, npair2, chunk2, 0)
        pltpu.sync_copy(tab, out_hbm.at[cid, pl.ds(r0, RPT), :])

    return scatter


def _finish_body(tx_ref, ty_ref, tz_ref, b2_ref, o_ref):
    agg = jnp.maximum(
        jnp.maximum(jnp.maximum(tx_ref[0], tx_ref[1]),
                    jnp.maximum(ty_ref[0], ty_ref[1])),
        jnp.maximum(tz_ref[0], tz_ref[1]))
    o_ref[...] = jnp.maximum(agg.T[:N, :] + b2_ref[...], 0.0)


def _finish(aggX, aggY, aggZ, b2):
    return pl.pallas_call(
        _finish_body,
        out_shape=jax.ShapeDtypeStruct((N, OUTC), jnp.float32),
    )(aggX, aggY, aggZ, b2)


_gather_x = _make_gather(EX)
_gather_y = _make_gather(EY)
_gather_z = _make_gather(EZ)
_scatter_x = _make_scatter(EX)
_scatter_y = _make_scatter(EY)
_scatter_z = _make_scatter(EZ)


def kernel(x, edge_index, gamma, beta, W1, b1, W2, b2):
    w1a = W1[:, :INC]
    w1b = W1[:, INC:]
    wdT = (w1a - w1b).T
    wbT = w1b.T
    w2T = W2.T
    zero = jnp.zeros((HID, OUTC), jnp.float32)
    wcomb = jnp.block([[w2T, zero], [zero, w2T]])
    src = edge_index[0]
    dst = edge_index[1]
    dst_x, dst_y, dst_z = dst[:EX], dst[EX:EX + EY], dst[EX + EY:]
    src_x, src_y, src_z = src[:EX], src[EX:EX + EY], src[EX + EY:]

    t_tab = _prep(x, gamma.reshape(1, -1), beta.reshape(1, -1),
                  wdT, wbT, b1.reshape(1, -1))
    h2x = _gather_x(t_tab, dst_x, src_x)
    mtx = _msg(h2x, wcomb)
    h2y = _gather_y(t_tab, dst_y, src_y)
    mty = _msg(h2y, wcomb)
    h2z = _gather_z(t_tab, dst_z, src_z)
    aggX = _scatter_x(mtx, dst_x)
    mtz = _msg(h2z, wcomb)
    aggY = _scatter_y(mty, dst_y)
    aggZ = _scatter_z(mtz, dst_z)
    return _finish(aggX, aggY, aggZ, b2.reshape(1, -1))


# speedup vs baseline: 1.0192x; 1.0192x over previous
import functools

import jax
import jax.numpy as jnp
from jax import lax
from jax.experimental import pallas as pl
from jax.experimental.pallas import tpu as pltpu
from jax.experimental.pallas import tpu_sc as plsc

N = 10000
E = 320000
INC = 128
OUTC = 128
HID = 64
EPS = 1e-5

NC = 2
NS = 16
NW = NC * NS
LANES = 16

EX = 192000
EY = E - EX
GCH = 200
SCH = 640
RPT = OUTC // NS
NP = 10240

PR = 1000
BE = 1280

_SC_MESH = plsc.VectorSubcoreMesh(
    core_axis_name="c", subcore_axis_name="s", num_cores=NC, num_subcores=NS
)


def _prep_body(x_ref, gamma_ref, beta_ref, wd_ref, wb_ref, b1_ref,
               t_ref, scale_ref, shift_ref):
    i = pl.program_id(0)

    @pl.when(i == 0)
    def _():
        xv = x_ref[...]
        mean = jnp.mean(xv, axis=0, keepdims=True)
        var = jnp.mean((xv - mean) ** 2, axis=0, keepdims=True)
        scale = gamma_ref[...] * lax.rsqrt(var + EPS)
        scale_ref[...] = scale
        shift_ref[...] = beta_ref[...] - mean * scale

    rows = x_ref[pl.ds(i * PR, PR), :]
    xb = rows * scale_ref[...] + shift_ref[...]
    a = (jnp.dot(xb, wd_ref[...], preferred_element_type=jnp.float32)
         + b1_ref[...])
    b = jnp.dot(xb, wb_ref[...], preferred_element_type=jnp.float32)
    t_ref[...] = jnp.concatenate([a, b], axis=-1)


def _prep(x, gamma, beta, wdT, wbT, b1):
    return pl.pallas_call(
        _prep_body,
        grid=(N // PR,),
        in_specs=[
            pl.BlockSpec((N, INC), lambda i: (0, 0)),
            pl.BlockSpec((1, INC), lambda i: (0, 0)),
            pl.BlockSpec((1, INC), lambda i: (0, 0)),
            pl.BlockSpec((INC, HID), lambda i: (0, 0)),
            pl.BlockSpec((INC, HID), lambda i: (0, 0)),
            pl.BlockSpec((1, HID), lambda i: (0, 0)),
        ],
        out_specs=pl.BlockSpec((PR, 2 * HID), lambda i: (i, 0)),
        out_shape=jax.ShapeDtypeStruct((N, 2 * HID), jnp.float32),
        scratch_shapes=[
            pltpu.VMEM((1, INC), jnp.float32),
            pltpu.VMEM((1, INC), jnp.float32),
        ],
    )(x, gamma, beta, wdT, wbT, b1)


def _make_gather(ne):
    npairs = ne // 2
    epw = npairs // NW
    nch = epw // GCH
    assert nch * GCH * NW * 2 == ne and (epw % 8) == 0

    @functools.partial(
        pl.kernel,
        out_type=jax.ShapeDtypeStruct((npairs, 2 * HID), jnp.float32),
        mesh=_SC_MESH,
        scratch_types=[
            pltpu.VMEM((GCH,), jnp.int32),
            pltpu.VMEM((GCH,), jnp.int32),
            pltpu.VMEM((GCH,), jnp.int32),
            pltpu.VMEM((GCH,), jnp.int32),
            pltpu.VMEM((GCH, 2 * HID), jnp.float32),
            pltpu.VMEM((GCH, 2 * HID), jnp.float32),
            pltpu.VMEM((GCH, 2 * HID), jnp.float32),
            pltpu.VMEM((GCH, 2 * HID), jnp.float32),
            pltpu.SemaphoreType.DMA,
            pltpu.SemaphoreType.DMA,
            pltpu.SemaphoreType.DMA,
        ],
    )
    def gather(t_hbm, dst_hbm, src_hbm, h_hbm,
               d0, s0, d1, s1, td0, ts0, td1, ts1, semi, semg0, semg1):
        wid = lax.axis_index("s") * NC + lax.axis_index("c")
        base = wid * epw

        def chunk(c, _):
            off = base + c * GCH
            ci0 = pltpu.async_copy(dst_hbm.at[pl.ds(off, GCH)], d0, semi)
            ci1 = pltpu.async_copy(src_hbm.at[pl.ds(off, GCH)], s0, semi)
            ci2 = pltpu.async_copy(dst_hbm.at[pl.ds(off + npairs, GCH)], d1,
                                   semi)
            ci3 = pltpu.async_copy(src_hbm.at[pl.ds(off + npairs, GCH)], s1,
                                   semi)
            ci0.wait()
            ci1.wait()
            ci2.wait()
            ci3.wait()
            g0d = pltpu.async_copy(t_hbm.at[d0], td0, semg0)
            g0s = pltpu.async_copy(t_hbm.at[s0], ts0, semg0)
            g1d = pltpu.async_copy(t_hbm.at[d1], td1, semg1)
            g1s = pltpu.async_copy(t_hbm.at[s1], ts1, semg1)
            g0d.wait()
            g0s.wait()

            def row0(e, _):
                for j in range(HID // LANES):
                    va = td0[e, pl.ds(j * LANES, LANES)]
                    vb = ts0[e, pl.ds(HID + j * LANES, LANES)]
                    td0[e, pl.ds(j * LANES, LANES)] = jnp.maximum(va + vb, 0.0)
                return 0

            lax.fori_loop(0, GCH, row0, 0)
            g1d.wait()
            g1s.wait()

            def row1(e, _):
                for j in range(HID // LANES):
                    va = td1[e, pl.ds(j * LANES, LANES)]
                    vb = ts1[e, pl.ds(HID + j * LANES, LANES)]
                    td0[e, pl.ds(HID + j * LANES, LANES)] = (
                        jnp.maximum(va + vb, 0.0))
                return 0

            lax.fori_loop(0, GCH, row1, 0)
            pltpu.sync_copy(td0, h_hbm.at[pl.ds(off, GCH)])
            return 0

        lax.fori_loop(0, nch, chunk, 0)

    return gather


def _msg_body(h_ref, wc_ref, m_ref):
    m = jnp.dot(h_ref[...], wc_ref[...], preferred_element_type=jnp.float32)
    m_ref[0] = m[:, :OUTC].T
    m_ref[1] = m[:, OUTC:].T


def _msg(h2, wcomb):
    npairs = h2.shape[0]
    return pl.pallas_call(
        _msg_body,
        grid=(npairs // BE,),
        in_specs=[
            pl.BlockSpec((BE, 2 * HID), lambda i: (i, 0)),
            pl.BlockSpec((2 * HID, 2 * OUTC), lambda i: (0, 0)),
        ],
        out_specs=pl.BlockSpec((2, OUTC, BE), lambda i: (0, 0, i)),
        out_shape=jax.ShapeDtypeStruct((2, OUTC, npairs), jnp.float32),
    )(h2, wcomb)


def _make_scatter(ne):
    nsc = ne // 2
    npair2 = nsc // (2 * SCH)
    assert npair2 * 2 * SCH == nsc

    @functools.partial(
        pl.kernel,
        out_type=jax.ShapeDtypeStruct((2, OUTC, NP), jnp.float32),
        mesh=_SC_MESH,
        scratch_types=[
            pltpu.VMEM((RPT, NP), jnp.float32),
            pltpu.VMEM((SCH,), jnp.int32),
            pltpu.VMEM((SCH,), jnp.int32),
            pltpu.VMEM((RPT, SCH), jnp.float32),
            pltpu.VMEM((RPT, SCH), jnp.float32),
            pltpu.SemaphoreType.DMA,
            pltpu.SemaphoreType.DMA,
        ],
        compiler_params=pltpu.CompilerParams(needs_layout_passes=False),
    )
    def scatter(mt_hbm, dst_hbm, out_hbm,
                tab, ibuf0, ibuf1, mbuf0, mbuf1, semA, semB):
        cid = lax.axis_index("c")
        sid = lax.axis_index("s")
        r0 = sid * RPT

        neg_inf = jnp.full((LANES,), -jnp.inf, jnp.float32)

        def initk(k, _):
            for r in range(RPT):
                tab[r, pl.ds(k * LANES, LANES)] = neg_inf
            return 0

        lax.fori_loop(0, NP // LANES, initk, 0)

        ibufs = (ibuf0, ibuf1)
        mbufs = (mbuf0, mbuf1)
        sems = (semA, semB)

        def consume(ibuf, mbuf):
            def group(g, _):
                idx = ibuf[pl.ds(g * LANES, LANES)]
                _, last = plsc.scan_count(idx)
                all_distinct = jnp.all(last)

                def fast():
                    for r in range(RPT):
                        rvec = jnp.full((LANES,), r, jnp.int32)
                        val = mbuf[r, pl.ds(g * LANES, LANES)]
                        cur = plsc.load_gather(tab, [rvec, idx])
                        plsc.store_scatter(tab, [rvec, idx],
                                           jnp.maximum(cur, val))

                def slow():
                    for r in range(RPT):
                        rvec = jnp.full((LANES,), r, jnp.int32)
                        val = mbuf[r, pl.ds(g * LANES, LANES)]

                        def attempt(_):
                            cur = plsc.load_gather(tab, [rvec, idx])
                            need = val > cur
                            plsc.store_scatter(tab, [rvec, idx], val,
                                               mask=need)
                            cur2 = plsc.load_gather(tab, [rvec, idx])
                            return jnp.any(val > cur2)

                        lax.while_loop(lambda p: p, attempt, attempt(True))

                lax.cond(all_distinct, fast, slow)
                return 0

            lax.fori_loop(0, SCH // LANES, group, 0)

        def chunk2(cc, _):
            handles = []
            for sl in range(2):
                off = (2 * cc + sl) * SCH
                handles.append((
                    pltpu.async_copy(
                        dst_hbm.at[pl.ds(cid * nsc + off, SCH)], ibufs[sl],
                        sems[sl]),
                    pltpu.async_copy(
                        mt_hbm.at[cid, pl.ds(r0, RPT), pl.ds(off, SCH)],
                        mbufs[sl], sems[sl]),
                ))
            for sl in range(2):
                handles[sl][0].wait()
                handles[sl][1].wait()
                consume(ibufs[sl], mbufs[sl])
            return 0

        lax.fori_loop(0, npair2, chunk2, 0)
        pltpu.sync_copy(tab, out_hbm.at[cid, pl.ds(r0, RPT), :])

    return scatter


def _finish_body(tx_ref, ty_ref, b2_ref, o_ref):
    agg = jnp.maximum(jnp.maximum(tx_ref[0], tx_ref[1]),
                      jnp.maximum(ty_ref[0], ty_ref[1]))
    o_ref[...] = jnp.maximum(agg.T[:N, :] + b2_ref[...], 0.0)


def _finish(aggX, aggY, b2):
    return pl.pallas_call(
        _finish_body,
        out_shape=jax.ShapeDtypeStruct((N, OUTC), jnp.float32),
    )(aggX, aggY, b2)


_gather_x = _make_gather(EX)
_gather_y = _make_gather(EY)
_scatter_x = _make_scatter(EX)
_scatter_y = _make_scatter(EY)


def kernel(x, edge_index, gamma, beta, W1, b1, W2, b2):
    w1a = W1[:, :INC]
    w1b = W1[:, INC:]
    wdT = (w1a - w1b).T
    wbT = w1b.T
    w2T = W2.T
    zero = jnp.zeros((HID, OUTC), jnp.float32)
    wcomb = jnp.block([[w2T, zero], [zero, w2T]])
    src = edge_index[0]
    dst = edge_index[1]
    dst_x, dst_y = dst[:EX], dst[EX:]
    src_x, src_y = src[:EX], src[EX:]

    t_tab = _prep(x, gamma.reshape(1, -1), beta.reshape(1, -1),
                  wdT, wbT, b1.reshape(1, -1))
    h2x = _gather_x(t_tab, dst_x, src_x)
    mtx = _msg(h2x, wcomb)
    h2y = _gather_y(t_tab, dst_y, src_y)
    aggX = _scatter_x(mtx, dst_x)
    mty = _msg(h2y, wcomb)
    aggY = _scatter_y(mty, dst_y)
    return _finish(aggX, aggY, b2.reshape(1, -1))
